# Initial kernel scaffold; baseline (speedup 1.0000x reference)
import jax
import jax.numpy as jnp
from jax.experimental import pallas as pl


def kernel(cfeats, cedge_feats, ctypes, tfeats, tedge_feats, c_src, c_dst, t_src, t_dst, op_emb, c_Ws, c_bs, t_Ws, t_bs, Wn, bn, Ws_final, bs_final):
    raise NotImplementedError("write your pallas kernel here")


# speedup vs baseline: 5.5072x; 5.5072x over previous
import functools

import jax
import jax.numpy as jnp
from jax import lax
from jax.experimental import pallas as pl
from jax.experimental.pallas import tpu as pltpu
from jax.experimental.pallas import tpu_sc as plsc

N = 10000
NPAD = 10240
E = 320000
HID = 512
BW = 64
C = 80
NB = 5
NS = 16
ROWS_PER_TILE = E // C // NS
NGROUPS = ROWS_PER_TILE // NB
NPT = NPAD // NS

_f32 = jnp.float32


def _zero_vmem(ref, nrows, ncols):
    z = jnp.zeros((16,), _f32)

    def body(i, _):
        for j in range(ncols // 16):
            ref[i, pl.ds(16 * j, 16)] = z
        return 0

    lax.fori_loop(0, nrows, body, 0)


def _fill_ones(ref, nrows, ncols):
    o = jnp.ones((16,), _f32)

    def body(i, _):
        for j in range(ncols // 16):
            ref[i, pl.ds(16 * j, 16)] = o
        return 0

    lax.fori_loop(0, nrows, body, 0)


def _mesh():
    return plsc.VectorSubcoreMesh(core_axis_name="c", subcore_axis_name="s",
                                  num_cores=2, num_subcores=NS)


@functools.lru_cache(maxsize=None)
def _build_deg_eagg():
    return functools.partial(
        pl.kernel,
        out_type=tuple(jax.ShapeDtypeStruct((NPAD, 16), _f32)
                       for _ in range(6)),
        mesh=_mesh(),
        scratch_types=(
            pltpu.VMEM((ROWS_PER_TILE, C), jnp.int32),
            pltpu.VMEM((ROWS_PER_TILE, C), jnp.int32),
            pltpu.VMEM((NB, C, 16), _f32),
            pltpu.VMEM((C, 16), _f32),
            pltpu.VMEM((NPT, 16), _f32),
            pltpu.VMEM_SHARED((NPAD, 16), _f32),
            pltpu.VMEM_SHARED((NPAD, 16), _f32),
            pltpu.VMEM_SHARED((NPAD, 16), _f32),
            pltpu.SemaphoreType.DMA,
        ),
        compiler_params=pltpu.CompilerParams(use_tc_tiling_on_sc=False),
    )(_sc_deg_eagg)


def _sc_deg_eagg(c_src2, c_dst2, cef, t_src2, t_dst2, tef,
                 od_c, id_c, ea_c, od_t, id_t, ea_t,
                 src_v, dst_v, ef_v, ones_v, zb, od_s, id_s, ea_s, sem):
    cid = lax.axis_index("c")
    sid = lax.axis_index("s")

    _fill_ones(ones_v, C, 16)
    _zero_vmem(zb, NPT, 16)

    r0 = sid * NPT
    pltpu.sync_copy(zb, od_s.at[pl.ds(r0, NPT)])
    pltpu.sync_copy(zb, id_s.at[pl.ds(r0, NPT)])
    pltpu.sync_copy(zb, ea_s.at[pl.ds(r0, NPT)])
    plsc.subcore_barrier()

    def run(src2, dst2, ef):
        base_row = sid * ROWS_PER_TILE
        pltpu.sync_copy(src2.at[sid], src_v)
        pltpu.sync_copy(dst2.at[sid], dst_v)

        def group(g, _):
            j0 = g * NB
            descs = [
                pltpu.async_copy(
                    ef.at[pl.ds((base_row + j0 + b) * C, C)], ef_v.at[b], sem)
                for b in range(NB)
            ]
            for b in range(NB):
                descs[b].wait()
                pltpu.sync_copy(ones_v, od_s.at[src_v.at[j0 + b]], add=True)
                pltpu.sync_copy(ones_v, id_s.at[dst_v.at[j0 + b]], add=True)
                pltpu.sync_copy(ef_v.at[b], ea_s.at[dst_v.at[j0 + b]], add=True)
            return 0

        lax.fori_loop(0, NGROUPS, group, 0)

    @pl.when(cid == 0)
    def _():
        run(c_src2, c_dst2, cef)

    @pl.when(cid == 1)
    def _():
        run(t_src2, t_dst2, tef)

    plsc.subcore_barrier()

    @pl.when(cid == 0)
    def _():
        pltpu.sync_copy(od_s.at[pl.ds(r0, NPT)], od_c.at[pl.ds(r0, NPT)])
        pltpu.sync_copy(id_s.at[pl.ds(r0, NPT)], id_c.at[pl.ds(r0, NPT)])
        pltpu.sync_copy(ea_s.at[pl.ds(r0, NPT)], ea_c.at[pl.ds(r0, NPT)])

    @pl.when(cid == 1)
    def _():
        pltpu.sync_copy(od_s.at[pl.ds(r0, NPT)], od_t.at[pl.ds(r0, NPT)])
        pltpu.sync_copy(id_s.at[pl.ds(r0, NPT)], id_t.at[pl.ds(r0, NPT)])
        pltpu.sync_copy(ea_s.at[pl.ds(r0, NPT)], ea_t.at[pl.ds(r0, NPT)])


@functools.lru_cache(maxsize=None)
def _make_segsum(W, ncb):
    out_t = tuple(jax.ShapeDtypeStruct((NPAD, W), _f32)
                  for _ in range(2 * ncb))

    @functools.partial(
        pl.kernel,
        out_type=out_t,
        mesh=_mesh(),
        scratch_types=(
            pltpu.VMEM((ROWS_PER_TILE, C), jnp.int32),
            pltpu.VMEM((ROWS_PER_TILE, C), jnp.int32),
            pltpu.VMEM((NB, C, W), _f32),
            pltpu.VMEM((NPT // 5, W), _f32),
            pltpu.VMEM_SHARED((NPAD, W), _f32),
            pltpu.SemaphoreType.DMA,
        ),
        compiler_params=pltpu.CompilerParams(use_tc_tiling_on_sc=False),
    )
    def segsum(*refs):
        fs = refs[0:2 * ncb]
        c_src2, c_dst2, t_src2, t_dst2 = refs[2 * ncb:2 * ncb + 4]
        outs = refs[2 * ncb + 4:4 * ncb + 4]
        src_v, dst_v, rows_v, zb, slab, sem = refs[4 * ncb + 4:]

        cid = lax.axis_index("c")
        sid = lax.axis_index("s")
        r0 = sid * NPT
        ZR = NPT // 5
        _zero_vmem(zb, ZR, W)

        def zero_own_range():
            for z in range(5):
                pltpu.sync_copy(zb, slab.at[pl.ds(r0 + z * ZR, ZR)])

        def run(src2, dst2, fs_blocks, out_blocks):
            pltpu.sync_copy(src2.at[sid], src_v)
            pltpu.sync_copy(dst2.at[sid], dst_v)
            zero_own_range()
            for cb in range(len(fs_blocks)):
                plsc.subcore_barrier()

                def group(g, _):
                    j0 = g * NB
                    descs = [
                        pltpu.async_copy(
                            fs_blocks[cb].at[src_v.at[j0 + b]],
                            rows_v.at[b], sem)
                        for b in range(NB)
                    ]
                    for b in range(NB):
                        descs[b].wait()
                        pltpu.sync_copy(
                            rows_v.at[b], slab.at[dst_v.at[j0 + b]], add=True)
                    return 0

                lax.fori_loop(0, NGROUPS, group, 0)
                plsc.subcore_barrier()
                pltpu.sync_copy(slab.at[pl.ds(r0, NPT)],
                                out_blocks[cb].at[pl.ds(r0, NPT)])
                if cb + 1 < len(fs_blocks):
                    zero_own_range()

        @pl.when(cid == 0)
        def _():
            run(c_src2, c_dst2, fs[:ncb], outs[:ncb])

        @pl.when(cid == 1)
        def _():
            run(t_src2, t_dst2, fs[ncb:], outs[ncb:])

    return segsum


RB = 1024
GRID = NPAD // RB


def _row_spec(w):
    return pl.BlockSpec((RB, w), lambda i: (i, 0))


def _full_spec(shape):
    return pl.BlockSpec(shape, lambda i: tuple(0 for _ in shape))


def _prep_kernel(odc, idc, odt, idt, cf, ct, emb, tf,
                 fs0ca, fs0cb, fs0ta, fs0tb, sic, soc, sit, sot):
    so_c = lax.rsqrt(jnp.maximum(odc[:, 0:1], 1.0))
    si_c = lax.rsqrt(jnp.maximum(idc[:, 0:1], 1.0))
    so_t = lax.rsqrt(jnp.maximum(odt[:, 0:1], 1.0))
    si_t = lax.rsqrt(jnp.maximum(idt[:, 0:1], 1.0))
    sic[...] = si_c
    soc[...] = so_c
    sit[...] = si_t
    sot[...] = so_t
    onehot = (ct[...] == lax.broadcasted_iota(jnp.int32, (RB, 64), 1))
    e = jnp.dot(onehot.astype(_f32), emb[...],
                preferred_element_type=_f32)
    fs0ca[...] = cf[...][:, 0:64] * so_c
    fs0cb[...] = jnp.concatenate([cf[...][:, 64:120], e], axis=1) * so_c
    fs0ta[...] = tf[...][:, 0:64] * so_t
    fs0tb[...] = tf[...][:, 64:128] * so_t


def _prep(odc, idc, odt, idt, cfeats, ctypes2, op_emb, tfeats):
    out = pl.pallas_call(
        _prep_kernel,
        grid=(GRID,),
        in_specs=[
            _row_spec(16), _row_spec(16), _row_spec(16), _row_spec(16),
            _row_spec(120), pl.BlockSpec((RB, 1), lambda i: (i, 0)),
            _full_spec((64, 8)), _row_spec(128),
        ],
        out_specs=[_row_spec(64)] * 4 + [
            pl.BlockSpec((RB, 1), lambda i: (i, 0)) for _ in range(4)],
        out_shape=[jax.ShapeDtypeStruct((NPAD, 64), _f32)] * 4 + [
            jax.ShapeDtypeStruct((NPAD, 1), _f32) for _ in range(4)],
    )(odc, idc, odt, idt, cfeats, ctypes2, op_emb, tfeats)
    return out


def _layer_body(nparts, wout, relu, emit_w, args):
    hs = args[:nparts]
    ea, si, so, wt, wb, b = args[nparts:nparts + 6]
    acc = jnp.dot(hs[0][...], wt[...][0:BW], preferred_element_type=_f32)
    for p in range(1, nparts):
        acc = acc + jnp.dot(hs[p][...], wt[...][BW * p:BW * (p + 1)],
                            preferred_element_type=_f32)
    acc = acc + jnp.dot(ea[...][:, 0:16], wb[...], preferred_element_type=_f32)
    y = acc * si[...] + b[...]
    if relu:
        y = jnp.maximum(y, 0.0)
    y = y * so[...]
    if emit_w is not None:
        return [jnp.dot(y, emit_w[...], preferred_element_type=_f32)]
    return [y[:, BW * cb:BW * (cb + 1)] for cb in range(wout // BW)]


def _make_layer_tc(nparts, win, wout, relu, emit_w_dim):
    nout = 1 if emit_w_dim else wout // BW
    ow = emit_w_dim if emit_w_dim else BW

    def kern(*refs):
        nargs = nparts + 6 + (1 if emit_w_dim else 0)
        ins, outs = refs[:2 * nargs], refs[2 * nargs:]
        for g in range(2):
            a = ins[g * nargs:(g + 1) * nargs]
            emit_w = a[nparts + 6] if emit_w_dim else None
            ys = _layer_body(nparts, wout, relu, emit_w, a)
            for k, y in enumerate(ys):
                outs[g * nout + k][...] = y

    def in_specs_one():
        sp = [_row_spec(BW) for _ in range(nparts)]
        sp += [_row_spec(16), pl.BlockSpec((RB, 1), lambda i: (i, 0)),
               pl.BlockSpec((RB, 1), lambda i: (i, 0)),
               _full_spec((win, wout)), _full_spec((16, wout)),
               _full_spec((1, wout))]
        if emit_w_dim:
            sp.append(_full_spec((wout, emit_w_dim)))
        return sp

    out_specs = [_row_spec(ow) for _ in range(2 * nout)]
    out_shape = [jax.ShapeDtypeStruct((NPAD, ow), _f32) for _ in range(2 * nout)]

    def call(c_args, t_args):
        return pl.pallas_call(
            kern,
            grid=(GRID,),
            in_specs=in_specs_one() + in_specs_one(),
            out_specs=out_specs,
            out_shape=out_shape,
        )(*c_args, *t_args)

    return call


_layer_ep0 = _make_layer_tc(2, 128, HID, True, None)
_layer_mid = _make_layer_tc(8, HID, HID, True, None)
_layer_ep3 = None


def _final_kernel(h5c, eac, sic, wbc, bc, wn, bnr,
                  h5t, eat, sit, wbt, bt, wsf, bsr, onccl, ostrat):
    c5 = (h5c[...] + jnp.dot(eac[...][:, 0:16], wbc[...],
                             preferred_element_type=_f32)) * sic[...] + bc[...]
    t5 = (h5t[...] + jnp.dot(eat[...][:, 0:16], wbt[...],
                             preferred_element_type=_f32)) * sit[...] + bt[...]
    onccl[...] = jnp.dot(c5, wn[...], preferred_element_type=_f32) + bnr[...]
    ostrat[...] = jnp.dot(t5, wsf[...], preferred_element_type=_f32) + bsr[...]


def _final(h5c, eac, sic, wbc, bc, wn, bn, h5t, eat, sit, wbt, bt, wsf, bs):
    return pl.pallas_call(
        _final_kernel,
        grid=(GRID,),
        in_specs=[
            _row_spec(64), _row_spec(16),
            pl.BlockSpec((RB, 1), lambda i: (i, 0)),
            _full_spec((16, 64)), _full_spec((1, 64)),
            _full_spec((64, 1)), _full_spec((1, 1)),
            _row_spec(64), _row_spec(16),
            pl.BlockSpec((RB, 1), lambda i: (i, 0)),
            _full_spec((16, 64)), _full_spec((1, 64)),
            _full_spec((64, 8)), _full_spec((1, 8)),
        ],
        out_specs=[pl.BlockSpec((RB, 1), lambda i: (i, 0)), _row_spec(8)],
        out_shape=[jax.ShapeDtypeStruct((NPAD, 1), _f32),
                   jax.ShapeDtypeStruct((NPAD, 8), _f32)],
    )(h5c, eac, sic, wbc, bc, wn, bn, h5t, eat, sit, wbt, bt, wsf, bs)


def kernel(cfeats, cedge_feats, ctypes, tfeats, tedge_feats, c_src, c_dst,
           t_src, t_dst, op_emb, c_Ws, c_bs, t_Ws, t_bs, Wn, bn, Ws_final,
           bs_final):
    global _layer_ep3
    if _layer_ep3 is None:
        _layer_ep3 = _make_layer_tc(8, HID, HID, True, 64)
    _segsum128 = _make_segsum(BW, 2)
    _segsum512 = _make_segsum(BW, 8)
    _segsum64 = _make_segsum(BW, 1)

    i32 = jnp.int32
    cs2 = c_src.astype(i32).reshape(NS, ROWS_PER_TILE, C)
    cd2 = c_dst.astype(i32).reshape(NS, ROWS_PER_TILE, C)
    ts2 = t_src.astype(i32).reshape(NS, ROWS_PER_TILE, C)
    td2 = t_dst.astype(i32).reshape(NS, ROWS_PER_TILE, C)
    idx = (cs2, cd2, ts2, td2)

    od_c, id_c, ea_c, od_t, id_t, ea_t = _build_deg_eagg()(
        cs2, cd2, cedge_feats, ts2, td2, tedge_feats)

    pad = ((0, NPAD - N), (0, 0))
    fs0ca, fs0cb, fs0ta, fs0tb, sic, soc, sit, sot = _prep(
        od_c, id_c, od_t, id_t, jnp.pad(cfeats, pad),
        jnp.pad(ctypes.astype(i32).reshape(N, 1), pad),
        op_emb, jnp.pad(tfeats, pad))

    h1 = _segsum128(fs0ca, fs0cb, fs0ta, fs0tb, *idx)
    cW, tW = c_Ws[0], t_Ws[0]
    fs1 = _layer_ep0(
        (*h1[0:2], ea_c, sic, soc, cW[0:128], cW[128:144],
         c_bs[0].reshape(1, HID)),
        (*h1[2:4], ea_t, sit, sot, tW[0:128], tW[128:144],
         t_bs[0].reshape(1, HID)))
    fs1c, fs1t = fs1[0:8], fs1[8:16]

    for i in (1, 2):
        hc = _segsum512(*fs1c, *fs1t, *idx)
        cW, tW = c_Ws[i], t_Ws[i]
        fs1 = _layer_mid(
            (*hc[0:8], ea_c, sic, soc, cW[0:HID], cW[HID:],
             c_bs[i].reshape(1, HID)),
            (*hc[8:16], ea_t, sit, sot, tW[0:HID], tW[HID:],
             t_bs[i].reshape(1, HID)))
        fs1c, fs1t = fs1[0:8], fs1[8:16]

    hc = _segsum512(*fs1c, *fs1t, *idx)
    cW, tW = c_Ws[3], t_Ws[3]
    zc, zt = _layer_ep3(
        (*hc[0:8], ea_c, sic, soc, cW[0:HID], cW[HID:],
         c_bs[3].reshape(1, HID), c_Ws[4][0:HID]),
        (*hc[8:16], ea_t, sit, sot, tW[0:HID], tW[HID:],
         t_bs[3].reshape(1, HID), t_Ws[4][0:HID]))

    h5c, h5t = _segsum64(zc, zt, *idx)
    onccl, ostrat = _final(
        h5c, ea_c, sic, c_Ws[4][HID:], c_bs[4].reshape(1, 64),
        Wn, (bn + jnp.zeros((1, 1), _f32)),
        h5t, ea_t, sit, t_Ws[4][HID:], t_bs[4].reshape(1, 64),
        Ws_final, bs_final.reshape(1, 8))
    return (onccl.reshape(NPAD)[:N], ostrat[:N])


# speedup vs baseline: 6.7991x; 1.2346x over previous
import functools

import jax
import jax.numpy as jnp
from jax import lax
from jax.experimental import pallas as pl
from jax.experimental.pallas import tpu as pltpu
from jax.experimental.pallas import tpu_sc as plsc

N = 10000
NPAD = 10240
E = 320000
HID = 512
BW = 64
C = 40
NB = 5
NS = 16
ROWS_PER_TILE = E // C // NS
NGROUPS = ROWS_PER_TILE // NB
NPT = NPAD // NS

_f32 = jnp.float32


def _zero_vmem(ref, nrows, ncols):
    z = jnp.zeros((16,), _f32)

    def body(i, _):
        for j in range(ncols // 16):
            ref[i, pl.ds(16 * j, 16)] = z
        return 0

    lax.fori_loop(0, nrows, body, 0)


def _fill_ones(ref, nrows, ncols):
    o = jnp.ones((16,), _f32)

    def body(i, _):
        for j in range(ncols // 16):
            ref[i, pl.ds(16 * j, 16)] = o
        return 0

    lax.fori_loop(0, nrows, body, 0)


def _mesh():
    return plsc.VectorSubcoreMesh(core_axis_name="c", subcore_axis_name="s",
                                  num_cores=2, num_subcores=NS)


@functools.lru_cache(maxsize=None)
def _build_deg_eagg():
    return functools.partial(
        pl.kernel,
        out_type=tuple(jax.ShapeDtypeStruct((NPAD, 16), _f32)
                       for _ in range(6)),
        mesh=_mesh(),
        scratch_types=(
            pltpu.VMEM((ROWS_PER_TILE, C), jnp.int32),
            pltpu.VMEM((ROWS_PER_TILE, C), jnp.int32),
            pltpu.VMEM((NB, C, 16), _f32),
            pltpu.VMEM((C, 16), _f32),
            pltpu.VMEM((NPT, 16), _f32),
            pltpu.VMEM_SHARED((NPAD, 16), _f32),
            pltpu.VMEM_SHARED((NPAD, 16), _f32),
            pltpu.VMEM_SHARED((NPAD, 16), _f32),
            pltpu.SemaphoreType.DMA,
        ),
        compiler_params=pltpu.CompilerParams(use_tc_tiling_on_sc=False),
    )(_sc_deg_eagg)


def _sc_deg_eagg(c_src2, c_dst2, cef, t_src2, t_dst2, tef,
                 od_c, id_c, ea_c, od_t, id_t, ea_t,
                 src_v, dst_v, ef_v, ones_v, zb, od_s, id_s, ea_s, sem):
    cid = lax.axis_index("c")
    sid = lax.axis_index("s")

    _fill_ones(ones_v, C, 16)
    _zero_vmem(zb, NPT, 16)

    r0 = sid * NPT
    pltpu.sync_copy(zb, od_s.at[pl.ds(r0, NPT)])
    pltpu.sync_copy(zb, id_s.at[pl.ds(r0, NPT)])
    pltpu.sync_copy(zb, ea_s.at[pl.ds(r0, NPT)])
    plsc.subcore_barrier()

    def run(src2, dst2, ef):
        base_row = sid * ROWS_PER_TILE
        pltpu.sync_copy(src2.at[sid], src_v)
        pltpu.sync_copy(dst2.at[sid], dst_v)

        def group(g, _):
            j0 = g * NB
            descs = [
                pltpu.async_copy(
                    ef.at[pl.ds((base_row + j0 + b) * C, C)], ef_v.at[b], sem)
                for b in range(NB)
            ]
            for b in range(NB):
                descs[b].wait()
                pltpu.sync_copy(ones_v, od_s.at[src_v.at[j0 + b]], add=True)
                pltpu.sync_copy(ones_v, id_s.at[dst_v.at[j0 + b]], add=True)
                pltpu.sync_copy(ef_v.at[b], ea_s.at[dst_v.at[j0 + b]], add=True)
            return 0

        lax.fori_loop(0, NGROUPS, group, 0)

    @pl.when(cid == 0)
    def _():
        run(c_src2, c_dst2, cef)

    @pl.when(cid == 1)
    def _():
        run(t_src2, t_dst2, tef)

    plsc.subcore_barrier()

    @pl.when(cid == 0)
    def _():
        pltpu.sync_copy(od_s.at[pl.ds(r0, NPT)], od_c.at[pl.ds(r0, NPT)])
        pltpu.sync_copy(id_s.at[pl.ds(r0, NPT)], id_c.at[pl.ds(r0, NPT)])
        pltpu.sync_copy(ea_s.at[pl.ds(r0, NPT)], ea_c.at[pl.ds(r0, NPT)])

    @pl.when(cid == 1)
    def _():
        pltpu.sync_copy(od_s.at[pl.ds(r0, NPT)], od_t.at[pl.ds(r0, NPT)])
        pltpu.sync_copy(id_s.at[pl.ds(r0, NPT)], id_t.at[pl.ds(r0, NPT)])
        pltpu.sync_copy(ea_s.at[pl.ds(r0, NPT)], ea_t.at[pl.ds(r0, NPT)])


@functools.lru_cache(maxsize=None)
def _make_segsum(W, ncb):
    out_t = tuple(jax.ShapeDtypeStruct((NPAD, W), _f32)
                  for _ in range(2 * ncb))

    @functools.partial(
        pl.kernel,
        out_type=out_t,
        mesh=_mesh(),
        scratch_types=(
            pltpu.VMEM((ROWS_PER_TILE, C), jnp.int32),
            pltpu.VMEM((ROWS_PER_TILE, C), jnp.int32),
            pltpu.VMEM((2 * NB, C, W), _f32),
            pltpu.VMEM((NPT // 5, W), _f32),
            pltpu.VMEM_SHARED((NPAD, W), _f32),
            pltpu.SemaphoreType.DMA,
            pltpu.SemaphoreType.DMA,
        ),
        compiler_params=pltpu.CompilerParams(use_tc_tiling_on_sc=False),
    )
    def segsum(*refs):
        fs = refs[0:2 * ncb]
        c_src2, c_dst2, t_src2, t_dst2 = refs[2 * ncb:2 * ncb + 4]
        outs = refs[2 * ncb + 4:4 * ncb + 4]
        src_v, dst_v, rows_v, zb, slab, gsem, ssem = refs[4 * ncb + 4:]

        cid = lax.axis_index("c")
        sid = lax.axis_index("s")
        r0 = sid * NPT
        ZR = NPT // 5
        _zero_vmem(zb, ZR, W)

        def zero_own_range():
            for z in range(5):
                pltpu.sync_copy(zb, slab.at[pl.ds(r0 + z * ZR, ZR)])

        def run(src2, dst2, fs_blocks, out_blocks):
            pltpu.sync_copy(src2.at[sid], src_v)
            pltpu.sync_copy(dst2.at[sid], dst_v)
            zero_own_range()
            for cb in range(len(fs_blocks)):
                fsb = fs_blocks[cb]

                def gath(row, slot):
                    pltpu.async_copy(fsb.at[src_v.at[row]], rows_v.at[slot],
                                     gsem)

                def scat(row, slot):
                    pltpu.async_copy(rows_v.at[slot],
                                     slab.at[dst_v.at[row]], ssem, add=True)

                def drain_g():
                    pltpu.make_async_copy(fsb.at[src_v.at[0]], rows_v.at[0],
                                          gsem).wait()

                def drain_s():
                    pltpu.make_async_copy(rows_v.at[0],
                                          slab.at[dst_v.at[0]], ssem).wait()

                plsc.subcore_barrier()
                for b in range(NB):
                    gath(b, b)

                def two_groups(i, _):
                    for half, (s_cur, s_oth) in ((0, (0, NB)), (1, (NB, 0))):
                        g0 = 2 * i + half
                        if half == 0:
                            @pl.when(i > 0)
                            def _():
                                for b in range(NB):
                                    drain_s()
                            for b in range(NB):
                                gath((g0 + 1) * NB + b, s_oth + b)
                        else:
                            for b in range(NB):
                                drain_s()

                            @pl.when(i < NGROUPS // 2 - 1)
                            def _():
                                for b in range(NB):
                                    gath((g0 + 1) * NB + b, s_oth + b)
                        for b in range(NB):
                            drain_g()
                            scat(g0 * NB + b, s_cur + b)
                    return 0

                lax.fori_loop(0, NGROUPS // 2, two_groups, 0)
                for b in range(NB):
                    drain_s()
                plsc.subcore_barrier()
                pltpu.sync_copy(slab.at[pl.ds(r0, NPT)],
                                out_blocks[cb].at[pl.ds(r0, NPT)])
                if cb + 1 < len(fs_blocks):
                    zero_own_range()

        @pl.when(cid == 0)
        def _():
            run(c_src2, c_dst2, fs[:ncb], outs[:ncb])

        @pl.when(cid == 1)
        def _():
            run(t_src2, t_dst2, fs[ncb:], outs[ncb:])

    return segsum


RB = 1024
GRID = NPAD // RB


def _row_spec(w):
    return pl.BlockSpec((RB, w), lambda i: (i, 0))


def _full_spec(shape):
    return pl.BlockSpec(shape, lambda i: tuple(0 for _ in shape))


def _prep_kernel(odc, idc, odt, idt, cf, ct, emb, tf,
                 fs0ca, fs0cb, fs0ta, fs0tb, sic, soc, sit, sot):
    so_c = lax.rsqrt(jnp.maximum(odc[:, 0:1], 1.0))
    si_c = lax.rsqrt(jnp.maximum(idc[:, 0:1], 1.0))
    so_t = lax.rsqrt(jnp.maximum(odt[:, 0:1], 1.0))
    si_t = lax.rsqrt(jnp.maximum(idt[:, 0:1], 1.0))
    sic[...] = si_c
    soc[...] = so_c
    sit[...] = si_t
    sot[...] = so_t
    onehot = (ct[...] == lax.broadcasted_iota(jnp.int32, (RB, 64), 1))
    e = jnp.dot(onehot.astype(_f32), emb[...],
                preferred_element_type=_f32)
    fs0ca[...] = cf[...][:, 0:64] * so_c
    fs0cb[...] = jnp.concatenate([cf[...][:, 64:120], e], axis=1) * so_c
    fs0ta[...] = tf[...][:, 0:64] * so_t
    fs0tb[...] = tf[...][:, 64:128] * so_t


def _prep(odc, idc, odt, idt, cfeats, ctypes2, op_emb, tfeats):
    out = pl.pallas_call(
        _prep_kernel,
        grid=(GRID,),
        in_specs=[
            _row_spec(16), _row_spec(16), _row_spec(16), _row_spec(16),
            _row_spec(120), pl.BlockSpec((RB, 1), lambda i: (i, 0)),
            _full_spec((64, 8)), _row_spec(128),
        ],
        out_specs=[_row_spec(64)] * 4 + [
            pl.BlockSpec((RB, 1), lambda i: (i, 0)) for _ in range(4)],
        out_shape=[jax.ShapeDtypeStruct((NPAD, 64), _f32)] * 4 + [
            jax.ShapeDtypeStruct((NPAD, 1), _f32) for _ in range(4)],
    )(odc, idc, odt, idt, cfeats, ctypes2, op_emb, tfeats)
    return out


def _layer_body(nparts, wout, relu, emit_w, args):
    hs = args[:nparts]
    ea, si, so, wt, wb, b = args[nparts:nparts + 6]
    acc = jnp.dot(hs[0][...], wt[...][0:BW], preferred_element_type=_f32)
    for p in range(1, nparts):
        acc = acc + jnp.dot(hs[p][...], wt[...][BW * p:BW * (p + 1)],
                            preferred_element_type=_f32)
    acc = acc + jnp.dot(ea[...][:, 0:16], wb[...], preferred_element_type=_f32)
    y = acc * si[...] + b[...]
    if relu:
        y = jnp.maximum(y, 0.0)
    y = y * so[...]
    if emit_w is not None:
        return [jnp.dot(y, emit_w[...], preferred_element_type=_f32)]
    return [y[:, BW * cb:BW * (cb + 1)] for cb in range(wout // BW)]


def _make_layer_tc(nparts, win, wout, relu, emit_w_dim):
    nout = 1 if emit_w_dim else wout // BW
    ow = emit_w_dim if emit_w_dim else BW

    def kern(*refs):
        nargs = nparts + 6 + (1 if emit_w_dim else 0)
        ins, outs = refs[:2 * nargs], refs[2 * nargs:]
        for g in range(2):
            a = ins[g * nargs:(g + 1) * nargs]
            emit_w = a[nparts + 6] if emit_w_dim else None
            ys = _layer_body(nparts, wout, relu, emit_w, a)
            for k, y in enumerate(ys):
                outs[g * nout + k][...] = y

    def in_specs_one():
        sp = [_row_spec(BW) for _ in range(nparts)]
        sp += [_row_spec(16), pl.BlockSpec((RB, 1), lambda i: (i, 0)),
               pl.BlockSpec((RB, 1), lambda i: (i, 0)),
               _full_spec((win, wout)), _full_spec((16, wout)),
               _full_spec((1, wout))]
        if emit_w_dim:
            sp.append(_full_spec((wout, emit_w_dim)))
        return sp

    out_specs = [_row_spec(ow) for _ in range(2 * nout)]
    out_shape = [jax.ShapeDtypeStruct((NPAD, ow), _f32) for _ in range(2 * nout)]

    def call(c_args, t_args):
        return pl.pallas_call(
            kern,
            grid=(GRID,),
            in_specs=in_specs_one() + in_specs_one(),
            out_specs=out_specs,
            out_shape=out_shape,
        )(*c_args, *t_args)

    return call


_layer_ep0 = _make_layer_tc(2, 128, HID, True, None)
_layer_mid = _make_layer_tc(8, HID, HID, True, None)
_layer_ep3 = None


def _final_kernel(h5c, eac, sic, wbc, bc, wn, bnr,
                  h5t, eat, sit, wbt, bt, wsf, bsr, onccl, ostrat):
    c5 = (h5c[...] + jnp.dot(eac[...][:, 0:16], wbc[...],
                             preferred_element_type=_f32)) * sic[...] + bc[...]
    t5 = (h5t[...] + jnp.dot(eat[...][:, 0:16], wbt[...],
                             preferred_element_type=_f32)) * sit[...] + bt[...]
    onccl[...] = jnp.dot(c5, wn[...], preferred_element_type=_f32) + bnr[...]
    ostrat[...] = jnp.dot(t5, wsf[...], preferred_element_type=_f32) + bsr[...]


def _final(h5c, eac, sic, wbc, bc, wn, bn, h5t, eat, sit, wbt, bt, wsf, bs):
    return pl.pallas_call(
        _final_kernel,
        grid=(GRID,),
        in_specs=[
            _row_spec(64), _row_spec(16),
            pl.BlockSpec((RB, 1), lambda i: (i, 0)),
            _full_spec((16, 64)), _full_spec((1, 64)),
            _full_spec((64, 1)), _full_spec((1, 1)),
            _row_spec(64), _row_spec(16),
            pl.BlockSpec((RB, 1), lambda i: (i, 0)),
            _full_spec((16, 64)), _full_spec((1, 64)),
            _full_spec((64, 8)), _full_spec((1, 8)),
        ],
        out_specs=[pl.BlockSpec((RB, 1), lambda i: (i, 0)), _row_spec(8)],
        out_shape=[jax.ShapeDtypeStruct((NPAD, 1), _f32),
                   jax.ShapeDtypeStruct((NPAD, 8), _f32)],
    )(h5c, eac, sic, wbc, bc, wn, bn, h5t, eat, sit, wbt, bt, wsf, bs)


def kernel(cfeats, cedge_feats, ctypes, tfeats, tedge_feats, c_src, c_dst,
           t_src, t_dst, op_emb, c_Ws, c_bs, t_Ws, t_bs, Wn, bn, Ws_final,
           bs_final):
    global _layer_ep3
    if _layer_ep3 is None:
        _layer_ep3 = _make_layer_tc(8, HID, HID, True, 64)
    _segsum128 = _make_segsum(BW, 2)
    _segsum512 = _make_segsum(BW, 8)
    _segsum64 = _make_segsum(BW, 1)

    i32 = jnp.int32
    cs2 = c_src.astype(i32).reshape(NS, ROWS_PER_TILE, C)
    cd2 = c_dst.astype(i32).reshape(NS, ROWS_PER_TILE, C)
    ts2 = t_src.astype(i32).reshape(NS, ROWS_PER_TILE, C)
    td2 = t_dst.astype(i32).reshape(NS, ROWS_PER_TILE, C)
    idx = (cs2, cd2, ts2, td2)

    od_c, id_c, ea_c, od_t, id_t, ea_t = _build_deg_eagg()(
        cs2, cd2, cedge_feats, ts2, td2, tedge_feats)

    pad = ((0, NPAD - N), (0, 0))
    fs0ca, fs0cb, fs0ta, fs0tb, sic, soc, sit, sot = _prep(
        od_c, id_c, od_t, id_t, jnp.pad(cfeats, pad),
        jnp.pad(ctypes.astype(i32).reshape(N, 1), pad),
        op_emb, jnp.pad(tfeats, pad))

    h1 = _segsum128(fs0ca, fs0cb, fs0ta, fs0tb, *idx)
    cW, tW = c_Ws[0], t_Ws[0]
    fs1 = _layer_ep0(
        (*h1[0:2], ea_c, sic, soc, cW[0:128], cW[128:144],
         c_bs[0].reshape(1, HID)),
        (*h1[2:4], ea_t, sit, sot, tW[0:128], tW[128:144],
         t_bs[0].reshape(1, HID)))
    fs1c, fs1t = fs1[0:8], fs1[8:16]

    for i in (1, 2):
        hc = _segsum512(*fs1c, *fs1t, *idx)
        cW, tW = c_Ws[i], t_Ws[i]
        fs1 = _layer_mid(
            (*hc[0:8], ea_c, sic, soc, cW[0:HID], cW[HID:],
             c_bs[i].reshape(1, HID)),
            (*hc[8:16], ea_t, sit, sot, tW[0:HID], tW[HID:],
             t_bs[i].reshape(1, HID)))
        fs1c, fs1t = fs1[0:8], fs1[8:16]

    hc = _segsum512(*fs1c, *fs1t, *idx)
    cW, tW = c_Ws[3], t_Ws[3]
    zc, zt = _layer_ep3(
        (*hc[0:8], ea_c, sic, soc, cW[0:HID], cW[HID:],
         c_bs[3].reshape(1, HID), c_Ws[4][0:HID]),
        (*hc[8:16], ea_t, sit, sot, tW[0:HID], tW[HID:],
         t_bs[3].reshape(1, HID), t_Ws[4][0:HID]))

    h5c, h5t = _segsum64(zc, zt, *idx)
    onccl, ostrat = _final(
        h5c, ea_c, sic, c_Ws[4][HID:], c_bs[4].reshape(1, 64),
        Wn, (bn + jnp.zeros((1, 1), _f32)),
        h5t, ea_t, sit, t_Ws[4][HID:], t_bs[4].reshape(1, 64),
        Ws_final, bs_final.reshape(1, 8))
    return (onccl.reshape(NPAD)[:N], ostrat[:N])


# speedup vs baseline: 6.8042x; 1.0008x over previous
import functools

import jax
import jax.numpy as jnp
from jax import lax
from jax.experimental import pallas as pl
from jax.experimental.pallas import tpu as pltpu
from jax.experimental.pallas import tpu_sc as plsc

N = 10000
NPAD = 10240
E = 320000
HID = 512
BW = 64
C = 40
NB = 5
NS = 16
ROWS_PER_TILE = E // C // NS
NGROUPS = ROWS_PER_TILE // NB
NPT = NPAD // NS

_f32 = jnp.float32
_bf16 = jnp.bfloat16


def _bdot(a, b):
    return jnp.dot(a.astype(_bf16), b.astype(_bf16),
                   preferred_element_type=_f32)


def _zero_vmem(ref, nrows, ncols):
    z = jnp.zeros((16,), _f32)

    def body(i, _):
        for j in range(ncols // 16):
            ref[i, pl.ds(16 * j, 16)] = z
        return 0

    lax.fori_loop(0, nrows, body, 0)


def _fill_ones(ref, nrows, ncols):
    o = jnp.ones((16,), _f32)

    def body(i, _):
        for j in range(ncols // 16):
            ref[i, pl.ds(16 * j, 16)] = o
        return 0

    lax.fori_loop(0, nrows, body, 0)


def _mesh():
    return plsc.VectorSubcoreMesh(core_axis_name="c", subcore_axis_name="s",
                                  num_cores=2, num_subcores=NS)


@functools.lru_cache(maxsize=None)
def _build_deg_eagg():
    return functools.partial(
        pl.kernel,
        out_type=tuple(jax.ShapeDtypeStruct((NPAD, 16), _f32)
                       for _ in range(6)),
        mesh=_mesh(),
        scratch_types=(
            pltpu.VMEM((ROWS_PER_TILE, C), jnp.int32),
            pltpu.VMEM((ROWS_PER_TILE, C), jnp.int32),
            pltpu.VMEM((NB, C, 16), _f32),
            pltpu.VMEM((C, 16), _f32),
            pltpu.VMEM((NPT, 16), _f32),
            pltpu.VMEM_SHARED((NPAD, 16), _f32),
            pltpu.VMEM_SHARED((NPAD, 16), _f32),
            pltpu.VMEM_SHARED((NPAD, 16), _f32),
            pltpu.SemaphoreType.DMA,
        ),
        compiler_params=pltpu.CompilerParams(use_tc_tiling_on_sc=False),
    )(_sc_deg_eagg)


def _sc_deg_eagg(c_src2, c_dst2, cef, t_src2, t_dst2, tef,
                 od_c, id_c, ea_c, od_t, id_t, ea_t,
                 src_v, dst_v, ef_v, ones_v, zb, od_s, id_s, ea_s, sem):
    cid = lax.axis_index("c")
    sid = lax.axis_index("s")

    _fill_ones(ones_v, C, 16)
    _zero_vmem(zb, NPT, 16)

    r0 = sid * NPT
    pltpu.sync_copy(zb, od_s.at[pl.ds(r0, NPT)])
    pltpu.sync_copy(zb, id_s.at[pl.ds(r0, NPT)])
    pltpu.sync_copy(zb, ea_s.at[pl.ds(r0, NPT)])
    plsc.subcore_barrier()

    def run(src2, dst2, ef):
        base_row = sid * ROWS_PER_TILE
        pltpu.sync_copy(src2.at[sid], src_v)
        pltpu.sync_copy(dst2.at[sid], dst_v)

        def group(g, _):
            j0 = g * NB
            descs = [
                pltpu.async_copy(
                    ef.at[pl.ds((base_row + j0 + b) * C, C)], ef_v.at[b], sem)
                for b in range(NB)
            ]
            for b in range(NB):
                descs[b].wait()
                pltpu.sync_copy(ones_v, od_s.at[src_v.at[j0 + b]], add=True)
                pltpu.sync_copy(ones_v, id_s.at[dst_v.at[j0 + b]], add=True)
                pltpu.sync_copy(ef_v.at[b], ea_s.at[dst_v.at[j0 + b]], add=True)
            return 0

        lax.fori_loop(0, NGROUPS, group, 0)

    @pl.when(cid == 0)
    def _():
        run(c_src2, c_dst2, cef)

    @pl.when(cid == 1)
    def _():
        run(t_src2, t_dst2, tef)

    plsc.subcore_barrier()

    @pl.when(cid == 0)
    def _():
        pltpu.sync_copy(od_s.at[pl.ds(r0, NPT)], od_c.at[pl.ds(r0, NPT)])
        pltpu.sync_copy(id_s.at[pl.ds(r0, NPT)], id_c.at[pl.ds(r0, NPT)])
        pltpu.sync_copy(ea_s.at[pl.ds(r0, NPT)], ea_c.at[pl.ds(r0, NPT)])

    @pl.when(cid == 1)
    def _():
        pltpu.sync_copy(od_s.at[pl.ds(r0, NPT)], od_t.at[pl.ds(r0, NPT)])
        pltpu.sync_copy(id_s.at[pl.ds(r0, NPT)], id_t.at[pl.ds(r0, NPT)])
        pltpu.sync_copy(ea_s.at[pl.ds(r0, NPT)], ea_t.at[pl.ds(r0, NPT)])


@functools.lru_cache(maxsize=None)
def _make_segsum(W, ncb):
    out_t = tuple(jax.ShapeDtypeStruct((NPAD, W), _f32)
                  for _ in range(2 * ncb))

    @functools.partial(
        pl.kernel,
        out_type=out_t,
        mesh=_mesh(),
        scratch_types=(
            pltpu.VMEM((ROWS_PER_TILE, C), jnp.int32),
            pltpu.VMEM((ROWS_PER_TILE, C), jnp.int32),
            pltpu.VMEM((2 * NB, C, W), _f32),
            pltpu.VMEM((NPT // 5, W), _f32),
            pltpu.VMEM_SHARED((NPAD, W), _f32),
            pltpu.SemaphoreType.DMA,
            pltpu.SemaphoreType.DMA,
        ),
        compiler_params=pltpu.CompilerParams(use_tc_tiling_on_sc=False),
    )
    def segsum(*refs):
        fs = refs[0:2 * ncb]
        c_src2, c_dst2, t_src2, t_dst2 = refs[2 * ncb:2 * ncb + 4]
        outs = refs[2 * ncb + 4:4 * ncb + 4]
        src_v, dst_v, rows_v, zb, slab, gsem, ssem = refs[4 * ncb + 4:]

        cid = lax.axis_index("c")
        sid = lax.axis_index("s")
        r0 = sid * NPT
        ZR = NPT // 5
        _zero_vmem(zb, ZR, W)

        def zero_own_range():
            for z in range(5):
                pltpu.sync_copy(zb, slab.at[pl.ds(r0 + z * ZR, ZR)])

        def run(src2, dst2, fs_blocks, out_blocks):
            pltpu.sync_copy(src2.at[sid], src_v)
            pltpu.sync_copy(dst2.at[sid], dst_v)
            zero_own_range()
            for cb in range(len(fs_blocks)):
                fsb = fs_blocks[cb]

                def gath(row, slot):
                    pltpu.async_copy(fsb.at[src_v.at[row]], rows_v.at[slot],
                                     gsem)

                def scat(row, slot):
                    pltpu.async_copy(rows_v.at[slot],
                                     slab.at[dst_v.at[row]], ssem, add=True)

                def drain_g():
                    pltpu.make_async_copy(fsb.at[src_v.at[0]], rows_v.at[0],
                                          gsem).wait()

                def drain_s():
                    pltpu.make_async_copy(rows_v.at[0],
                                          slab.at[dst_v.at[0]], ssem).wait()

                plsc.subcore_barrier()
                for b in range(NB):
                    gath(b, b)

                def two_groups(i, _):
                    for half, (s_cur, s_oth) in ((0, (0, NB)), (1, (NB, 0))):
                        g0 = 2 * i + half
                        if half == 0:
                            @pl.when(i > 0)
                            def _():
                                for b in range(NB):
                                    drain_s()
                            for b in range(NB):
                                gath((g0 + 1) * NB + b, s_oth + b)
                        else:
                            for b in range(NB):
                                drain_s()

                            @pl.when(i < NGROUPS // 2 - 1)
                            def _():
                                for b in range(NB):
                                    gath((g0 + 1) * NB + b, s_oth + b)
                        for b in range(NB):
                            drain_g()
                            scat(g0 * NB + b, s_cur + b)
                    return 0

                lax.fori_loop(0, NGROUPS // 2, two_groups, 0)
                for b in range(NB):
                    drain_s()
                plsc.subcore_barrier()
                pltpu.sync_copy(slab.at[pl.ds(r0, NPT)],
                                out_blocks[cb].at[pl.ds(r0, NPT)])
                if cb + 1 < len(fs_blocks):
                    zero_own_range()

        @pl.when(cid == 0)
        def _():
            run(c_src2, c_dst2, fs[:ncb], outs[:ncb])

        @pl.when(cid == 1)
        def _():
            run(t_src2, t_dst2, fs[ncb:], outs[ncb:])

    return segsum


RB = 1024
GRID = NPAD // RB


def _row_spec(w):
    return pl.BlockSpec((RB, w), lambda i: (i, 0))


def _full_spec(shape):
    return pl.BlockSpec(shape, lambda i: tuple(0 for _ in shape))


def _prep_kernel(odc, idc, odt, idt, cf, ct, emb, tf,
                 fs0ca, fs0cb, fs0ta, fs0tb, sic, soc, sit, sot):
    so_c = lax.rsqrt(jnp.maximum(odc[:, 0:1], 1.0))
    si_c = lax.rsqrt(jnp.maximum(idc[:, 0:1], 1.0))
    so_t = lax.rsqrt(jnp.maximum(odt[:, 0:1], 1.0))
    si_t = lax.rsqrt(jnp.maximum(idt[:, 0:1], 1.0))
    sic[...] = si_c
    soc[...] = so_c
    sit[...] = si_t
    sot[...] = so_t
    onehot = (ct[...] == lax.broadcasted_iota(jnp.int32, (RB, 64), 1))
    e = _bdot(onehot.astype(_f32), emb[...])
    fs0ca[...] = cf[...][:, 0:64] * so_c
    fs0cb[...] = jnp.concatenate([cf[...][:, 64:120], e], axis=1) * so_c
    fs0ta[...] = tf[...][:, 0:64] * so_t
    fs0tb[...] = tf[...][:, 64:128] * so_t


def _prep(odc, idc, odt, idt, cfeats, ctypes2, op_emb, tfeats):
    out = pl.pallas_call(
        _prep_kernel,
        grid=(GRID,),
        in_specs=[
            _row_spec(16), _row_spec(16), _row_spec(16), _row_spec(16),
            _row_spec(120), pl.BlockSpec((RB, 1), lambda i: (i, 0)),
            _full_spec((64, 8)), _row_spec(128),
        ],
        out_specs=[_row_spec(64)] * 4 + [
            pl.BlockSpec((RB, 1), lambda i: (i, 0)) for _ in range(4)],
        out_shape=[jax.ShapeDtypeStruct((NPAD, 64), _f32)] * 4 + [
            jax.ShapeDtypeStruct((NPAD, 1), _f32) for _ in range(4)],
    )(odc, idc, odt, idt, cfeats, ctypes2, op_emb, tfeats)
    return out


def _layer_body(nparts, wout, relu, emit_w, args):
    hs = args[:nparts]
    ea, si, so, wt, wb, b = args[nparts:nparts + 6]
    acc = _bdot(hs[0][...], wt[...][0:BW])
    for p in range(1, nparts):
        acc = acc + _bdot(hs[p][...], wt[...][BW * p:BW * (p + 1)])
    acc = acc + _bdot(ea[...][:, 0:16], wb[...])
    y = acc * si[...] + b[...]
    if relu:
        y = jnp.maximum(y, 0.0)
    y = y * so[...]
    if emit_w is not None:
        return [_bdot(y, emit_w[...])]
    return [y[:, BW * cb:BW * (cb + 1)] for cb in range(wout // BW)]


def _make_layer_tc(nparts, win, wout, relu, emit_w_dim):
    nout = 1 if emit_w_dim else wout // BW
    ow = emit_w_dim if emit_w_dim else BW

    def kern(*refs):
        nargs = nparts + 6 + (1 if emit_w_dim else 0)
        ins, outs = refs[:2 * nargs], refs[2 * nargs:]
        for g in range(2):
            a = ins[g * nargs:(g + 1) * nargs]
            emit_w = a[nparts + 6] if emit_w_dim else None
            ys = _layer_body(nparts, wout, relu, emit_w, a)
            for k, y in enumerate(ys):
                outs[g * nout + k][...] = y

    def in_specs_one():
        sp = [_row_spec(BW) for _ in range(nparts)]
        sp += [_row_spec(16), pl.BlockSpec((RB, 1), lambda i: (i, 0)),
               pl.BlockSpec((RB, 1), lambda i: (i, 0)),
               _full_spec((win, wout)), _full_spec((16, wout)),
               _full_spec((1, wout))]
        if emit_w_dim:
            sp.append(_full_spec((wout, emit_w_dim)))
        return sp

    out_specs = [_row_spec(ow) for _ in range(2 * nout)]
    out_shape = [jax.ShapeDtypeStruct((NPAD, ow), _f32) for _ in range(2 * nout)]

    def call(c_args, t_args):
        return pl.pallas_call(
            kern,
            grid=(GRID,),
            in_specs=in_specs_one() + in_specs_one(),
            out_specs=out_specs,
            out_shape=out_shape,
        )(*c_args, *t_args)

    return call


_layer_ep0 = _make_layer_tc(2, 128, HID, True, None)
_layer_mid = _make_layer_tc(8, HID, HID, True, None)
_layer_ep3 = None


def _final_kernel(h5c, eac, sic, wbc, bc, wn, bnr,
                  h5t, eat, sit, wbt, bt, wsf, bsr, onccl, ostrat):
    c5 = (h5c[...] + _bdot(eac[...][:, 0:16], wbc[...])) * sic[...] + bc[...]
    t5 = (h5t[...] + _bdot(eat[...][:, 0:16], wbt[...])) * sit[...] + bt[...]
    onccl[...] = _bdot(c5, wn[...]) + bnr[...]
    ostrat[...] = _bdot(t5, wsf[...]) + bsr[...]


def _final(h5c, eac, sic, wbc, bc, wn, bn, h5t, eat, sit, wbt, bt, wsf, bs):
    return pl.pallas_call(
        _final_kernel,
        grid=(GRID,),
        in_specs=[
            _row_spec(64), _row_spec(16),
            pl.BlockSpec((RB, 1), lambda i: (i, 0)),
            _full_spec((16, 64)), _full_spec((1, 64)),
            _full_spec((64, 1)), _full_spec((1, 1)),
            _row_spec(64), _row_spec(16),
            pl.BlockSpec((RB, 1), lambda i: (i, 0)),
            _full_spec((16, 64)), _full_spec((1, 64)),
            _full_spec((64, 8)), _full_spec((1, 8)),
        ],
        out_specs=[pl.BlockSpec((RB, 1), lambda i: (i, 0)), _row_spec(8)],
        out_shape=[jax.ShapeDtypeStruct((NPAD, 1), _f32),
                   jax.ShapeDtypeStruct((NPAD, 8), _f32)],
    )(h5c, eac, sic, wbc, bc, wn, bn, h5t, eat, sit, wbt, bt, wsf, bs)


def kernel(cfeats, cedge_feats, ctypes, tfeats, tedge_feats, c_src, c_dst,
           t_src, t_dst, op_emb, c_Ws, c_bs, t_Ws, t_bs, Wn, bn, Ws_final,
           bs_final):
    global _layer_ep3
    if _layer_ep3 is None:
        _layer_ep3 = _make_layer_tc(8, HID, HID, True, 64)
    _segsum128 = _make_segsum(BW, 2)
    _segsum512 = _make_segsum(BW, 8)
    _segsum64 = _make_segsum(BW, 1)

    i32 = jnp.int32
    cs2 = c_src.astype(i32).reshape(NS, ROWS_PER_TILE, C)
    cd2 = c_dst.astype(i32).reshape(NS, ROWS_PER_TILE, C)
    ts2 = t_src.astype(i32).reshape(NS, ROWS_PER_TILE, C)
    td2 = t_dst.astype(i32).reshape(NS, ROWS_PER_TILE, C)
    idx = (cs2, cd2, ts2, td2)

    od_c, id_c, ea_c, od_t, id_t, ea_t = _build_deg_eagg()(
        cs2, cd2, cedge_feats, ts2, td2, tedge_feats)

    pad = ((0, NPAD - N), (0, 0))
    fs0ca, fs0cb, fs0ta, fs0tb, sic, soc, sit, sot = _prep(
        od_c, id_c, od_t, id_t, jnp.pad(cfeats, pad),
        jnp.pad(ctypes.astype(i32).reshape(N, 1), pad),
        op_emb, jnp.pad(tfeats, pad))

    h1 = _segsum128(fs0ca, fs0cb, fs0ta, fs0tb, *idx)
    cW, tW = c_Ws[0], t_Ws[0]
    fs1 = _layer_ep0(
        (*h1[0:2], ea_c, sic, soc, cW[0:128], cW[128:144],
         c_bs[0].reshape(1, HID)),
        (*h1[2:4], ea_t, sit, sot, tW[0:128], tW[128:144],
         t_bs[0].reshape(1, HID)))
    fs1c, fs1t = fs1[0:8], fs1[8:16]

    for i in (1, 2):
        hc = _segsum512(*fs1c, *fs1t, *idx)
        cW, tW = c_Ws[i], t_Ws[i]
        fs1 = _layer_mid(
            (*hc[0:8], ea_c, sic, soc, cW[0:HID], cW[HID:],
             c_bs[i].reshape(1, HID)),
            (*hc[8:16], ea_t, sit, sot, tW[0:HID], tW[HID:],
             t_bs[i].reshape(1, HID)))
        fs1c, fs1t = fs1[0:8], fs1[8:16]

    hc = _segsum512(*fs1c, *fs1t, *idx)
    cW, tW = c_Ws[3], t_Ws[3]
    zc, zt = _layer_ep3(
        (*hc[0:8], ea_c, sic, soc, cW[0:HID], cW[HID:],
         c_bs[3].reshape(1, HID), c_Ws[4][0:HID]),
        (*hc[8:16], ea_t, sit, sot, tW[0:HID], tW[HID:],
         t_bs[3].reshape(1, HID), t_Ws[4][0:HID]))

    h5c, h5t = _segsum64(zc, zt, *idx)
    onccl, ostrat = _final(
        h5c, ea_c, sic, c_Ws[4][HID:], c_bs[4].reshape(1, 64),
        Wn, (bn + jnp.zeros((1, 1), _f32)),
        h5t, ea_t, sit, t_Ws[4][HID:], t_bs[4].reshape(1, 64),
        Ws_final, bs_final.reshape(1, 8))
    return (onccl.reshape(NPAD)[:N], ostrat[:N])


# speedup vs baseline: 7.4119x; 1.0893x over previous
import functools

import jax
import jax.numpy as jnp
from jax import lax
from jax.experimental import pallas as pl
from jax.experimental.pallas import tpu as pltpu
from jax.experimental.pallas import tpu_sc as plsc

N = 10000
NPAD = 10240
E = 320000
HID = 512
BW = 64
C = 40
NB = 5
NS = 16
ROWS_PER_TILE = E // C // NS
NGROUPS = ROWS_PER_TILE // NB
NPT = NPAD // NS

_f32 = jnp.float32
_bf16 = jnp.bfloat16


def _bdot(a, b):
    return jnp.dot(a.astype(_bf16), b.astype(_bf16),
                   preferred_element_type=_f32)


def _zero_vmem(ref, nrows, ncols):
    z = jnp.zeros((16,), _f32)

    def body(i, _):
        for j in range(ncols // 16):
            ref[i, pl.ds(16 * j, 16)] = z
        return 0

    lax.fori_loop(0, nrows, body, 0)


def _fill_ones(ref, nrows, ncols):
    o = jnp.ones((16,), _f32)

    def body(i, _):
        for j in range(ncols // 16):
            ref[i, pl.ds(16 * j, 16)] = o
        return 0

    lax.fori_loop(0, nrows, body, 0)


def _mesh():
    return plsc.VectorSubcoreMesh(core_axis_name="c", subcore_axis_name="s",
                                  num_cores=2, num_subcores=NS)


@functools.lru_cache(maxsize=None)
def _build_deg_eagg():
    return functools.partial(
        pl.kernel,
        out_type=tuple(jax.ShapeDtypeStruct((NPAD, 16), _f32)
                       for _ in range(6)),
        mesh=_mesh(),
        scratch_types=(
            pltpu.VMEM((ROWS_PER_TILE, C), jnp.int32),
            pltpu.VMEM((ROWS_PER_TILE, C), jnp.int32),
            pltpu.VMEM((NB, C, 16), _f32),
            pltpu.VMEM((C, 16), _f32),
            pltpu.VMEM((NPT, 16), _f32),
            pltpu.VMEM_SHARED((NPAD, 16), _f32),
            pltpu.VMEM_SHARED((NPAD, 16), _f32),
            pltpu.VMEM_SHARED((NPAD, 16), _f32),
            pltpu.SemaphoreType.DMA,
        ),
        compiler_params=pltpu.CompilerParams(use_tc_tiling_on_sc=False),
    )(_sc_deg_eagg)


def _sc_deg_eagg(c_src2, c_dst2, cef, t_src2, t_dst2, tef,
                 od_c, id_c, ea_c, od_t, id_t, ea_t,
                 src_v, dst_v, ef_v, ones_v, zb, od_s, id_s, ea_s, sem):
    cid = lax.axis_index("c")
    sid = lax.axis_index("s")

    _fill_ones(ones_v, C, 16)
    _zero_vmem(zb, NPT, 16)

    r0 = sid * NPT
    pltpu.sync_copy(zb, od_s.at[pl.ds(r0, NPT)])
    pltpu.sync_copy(zb, id_s.at[pl.ds(r0, NPT)])
    pltpu.sync_copy(zb, ea_s.at[pl.ds(r0, NPT)])
    plsc.subcore_barrier()

    def run(src2, dst2, ef):
        base_row = sid * ROWS_PER_TILE
        pltpu.sync_copy(src2.at[sid], src_v)
        pltpu.sync_copy(dst2.at[sid], dst_v)

        def group(g, _):
            j0 = g * NB
            descs = [
                pltpu.async_copy(
                    ef.at[pl.ds((base_row + j0 + b) * C, C)], ef_v.at[b], sem)
                for b in range(NB)
            ]
            for b in range(NB):
                descs[b].wait()
                pltpu.sync_copy(ones_v, od_s.at[src_v.at[j0 + b]], add=True)
                pltpu.sync_copy(ones_v, id_s.at[dst_v.at[j0 + b]], add=True)
                pltpu.sync_copy(ef_v.at[b], ea_s.at[dst_v.at[j0 + b]], add=True)
            return 0

        lax.fori_loop(0, NGROUPS, group, 0)

    @pl.when(cid == 0)
    def _():
        run(c_src2, c_dst2, cef)

    @pl.when(cid == 1)
    def _():
        run(t_src2, t_dst2, tef)

    plsc.subcore_barrier()

    @pl.when(cid == 0)
    def _():
        pltpu.sync_copy(od_s.at[pl.ds(r0, NPT)], od_c.at[pl.ds(r0, NPT)])
        pltpu.sync_copy(id_s.at[pl.ds(r0, NPT)], id_c.at[pl.ds(r0, NPT)])
        pltpu.sync_copy(ea_s.at[pl.ds(r0, NPT)], ea_c.at[pl.ds(r0, NPT)])

    @pl.when(cid == 1)
    def _():
        pltpu.sync_copy(od_s.at[pl.ds(r0, NPT)], od_t.at[pl.ds(r0, NPT)])
        pltpu.sync_copy(id_s.at[pl.ds(r0, NPT)], id_t.at[pl.ds(r0, NPT)])
        pltpu.sync_copy(ea_s.at[pl.ds(r0, NPT)], ea_t.at[pl.ds(r0, NPT)])


@functools.lru_cache(maxsize=None)
def _make_segsum(W, nblocks, edge_split):
    nout = 2 if edge_split else nblocks
    rpt = ROWS_PER_TILE // 2 if edge_split else ROWS_PER_TILE
    ngroups = rpt // NB
    assert ngroups % 2 == 0
    out_t = tuple(jax.ShapeDtypeStruct((NPAD, W), _f32) for _ in range(nout))

    @functools.partial(
        pl.kernel,
        out_type=out_t,
        mesh=_mesh(),
        scratch_types=(
            pltpu.VMEM((rpt, C), jnp.int32),
            pltpu.VMEM((rpt, C), jnp.int32),
            pltpu.VMEM((2 * NB, C, W), _f32),
            pltpu.VMEM((NPT // 5, W), _f32),
            pltpu.VMEM_SHARED((NPAD, W), _f32),
            pltpu.SemaphoreType.DMA,
            pltpu.SemaphoreType.DMA,
        ),
        compiler_params=pltpu.CompilerParams(use_tc_tiling_on_sc=False),
    )
    def segsum(*refs):
        fs = refs[0:nblocks]
        src2, dst2 = refs[nblocks:nblocks + 2]
        outs = refs[nblocks + 2:nblocks + 2 + nout]
        src_v, dst_v, rows_v, zb, slab, gsem, ssem = refs[nblocks + 2 + nout:]

        cid = lax.axis_index("c")
        sid = lax.axis_index("s")
        r0 = sid * NPT
        ZR = NPT // 5
        _zero_vmem(zb, ZR, W)

        def zero_own_range():
            for z in range(5):
                pltpu.sync_copy(zb, slab.at[pl.ds(r0 + z * ZR, ZR)])

        def run(fs_blocks, out_blocks):
            if edge_split:
                pltpu.sync_copy(src2.at[cid, sid], src_v)
                pltpu.sync_copy(dst2.at[cid, sid], dst_v)
            else:
                pltpu.sync_copy(src2.at[sid], src_v)
                pltpu.sync_copy(dst2.at[sid], dst_v)
            zero_own_range()
            for cb in range(len(fs_blocks)):
                fsb = fs_blocks[cb]

                def gath(row, slot):
                    pltpu.async_copy(fsb.at[src_v.at[row]], rows_v.at[slot],
                                     gsem)

                def scat(row, slot):
                    pltpu.async_copy(rows_v.at[slot],
                                     slab.at[dst_v.at[row]], ssem, add=True)

                def drain_g():
                    pltpu.make_async_copy(fsb.at[src_v.at[0]], rows_v.at[0],
                                          gsem).wait()

                def drain_s():
                    pltpu.make_async_copy(rows_v.at[0],
                                          slab.at[dst_v.at[0]], ssem).wait()

                plsc.subcore_barrier()
                for b in range(NB):
                    gath(b, b)

                def two_groups(i, _):
                    for half, (s_cur, s_oth) in ((0, (0, NB)), (1, (NB, 0))):
                        g0 = 2 * i + half
                        if half == 0:
                            @pl.when(i > 0)
                            def _():
                                for b in range(NB):
                                    drain_s()
                            for b in range(NB):
                                gath((g0 + 1) * NB + b, s_oth + b)
                        else:
                            for b in range(NB):
                                drain_s()

                            @pl.when(i < ngroups // 2 - 1)
                            def _():
                                for b in range(NB):
                                    gath((g0 + 1) * NB + b, s_oth + b)
                        for b in range(NB):
                            drain_g()
                            scat(g0 * NB + b, s_cur + b)
                    return 0

                lax.fori_loop(0, ngroups // 2, two_groups, 0)
                for b in range(NB):
                    drain_s()
                plsc.subcore_barrier()
                pltpu.sync_copy(slab.at[pl.ds(r0, NPT)],
                                out_blocks[cb].at[pl.ds(r0, NPT)])
                if cb + 1 < len(fs_blocks):
                    zero_own_range()

        if edge_split:
            @pl.when(cid == 0)
            def _():
                run([fs[0]], [outs[0]])

            @pl.when(cid == 1)
            def _():
                run([fs[0]], [outs[1]])
        else:
            h = nblocks // 2

            @pl.when(cid == 0)
            def _():
                run(fs[:h], outs[:h])

            @pl.when(cid == 1)
            def _():
                run(fs[h:], outs[h:])

    return segsum


RB = 1024
GRID = NPAD // RB


def _row_spec(w):
    return pl.BlockSpec((RB, w), lambda i: (i, 0))


def _full_spec(shape):
    return pl.BlockSpec(shape, lambda i: tuple(0 for _ in shape))


def _prep_kernel(odc, idc, odt, idt, cf, ct, emb, tf,
                 fs0ca, fs0cb, fs0ta, fs0tb, sic, soc, sit, sot):
    so_c = lax.rsqrt(jnp.maximum(odc[:, 0:1], 1.0))
    si_c = lax.rsqrt(jnp.maximum(idc[:, 0:1], 1.0))
    so_t = lax.rsqrt(jnp.maximum(odt[:, 0:1], 1.0))
    si_t = lax.rsqrt(jnp.maximum(idt[:, 0:1], 1.0))
    sic[...] = si_c
    soc[...] = so_c
    sit[...] = si_t
    sot[...] = so_t
    onehot = (ct[...] == lax.broadcasted_iota(jnp.int32, (RB, 64), 1))
    e = _bdot(onehot.astype(_f32), emb[...])
    fs0ca[...] = cf[...][:, 0:64] * so_c
    fs0cb[...] = jnp.concatenate([cf[...][:, 64:120], e], axis=1) * so_c
    fs0ta[...] = tf[...][:, 0:64] * so_t
    fs0tb[...] = tf[...][:, 64:128] * so_t


def _prep(odc, idc, odt, idt, cfeats, ctypes2, op_emb, tfeats):
    out = pl.pallas_call(
        _prep_kernel,
        grid=(GRID,),
        in_specs=[
            _row_spec(16), _row_spec(16), _row_spec(16), _row_spec(16),
            _row_spec(120), pl.BlockSpec((RB, 1), lambda i: (i, 0)),
            _full_spec((64, 8)), _row_spec(128),
        ],
        out_specs=[_row_spec(64)] * 4 + [
            pl.BlockSpec((RB, 1), lambda i: (i, 0)) for _ in range(4)],
        out_shape=[jax.ShapeDtypeStruct((NPAD, 64), _f32)] * 4 + [
            jax.ShapeDtypeStruct((NPAD, 1), _f32) for _ in range(4)],
    )(odc, idc, odt, idt, cfeats, ctypes2, op_emb, tfeats)
    return out


def _layer_body(nparts, wout, relu, emit_w, args):
    hs = args[:nparts]
    ea, si, so, wt, wb, b = args[nparts:nparts + 6]
    acc = _bdot(hs[0][...], wt[...][0:BW])
    for p in range(1, nparts):
        acc = acc + _bdot(hs[p][...], wt[...][BW * p:BW * (p + 1)])
    acc = acc + _bdot(ea[...][:, 0:16], wb[...])
    y = acc * si[...] + b[...]
    if relu:
        y = jnp.maximum(y, 0.0)
    y = y * so[...]
    if emit_w is not None:
        return [_bdot(y, emit_w[...])]
    return [y[:, BW * cb:BW * (cb + 1)] for cb in range(wout // BW)]


@functools.lru_cache(maxsize=None)
def _make_layer_tc(nparts, win, wout, relu, emit_w_dim):
    nout = 1 if emit_w_dim else wout // BW
    ow = emit_w_dim if emit_w_dim else BW
    nargs = nparts + 6 + (1 if emit_w_dim else 0)

    def kern(*refs):
        ins, outs = refs[:nargs], refs[nargs:]
        emit_w = ins[nparts + 6] if emit_w_dim else None
        ys = _layer_body(nparts, wout, relu, emit_w, ins)
        for k, y in enumerate(ys):
            outs[k][...] = y

    def in_specs_one():
        sp = [_row_spec(BW) for _ in range(nparts)]
        sp += [_row_spec(16), pl.BlockSpec((RB, 1), lambda i: (i, 0)),
               pl.BlockSpec((RB, 1), lambda i: (i, 0)),
               _full_spec((win, wout)), _full_spec((16, wout)),
               _full_spec((1, wout))]
        if emit_w_dim:
            sp.append(_full_spec((wout, emit_w_dim)))
        return sp

    out_specs = [_row_spec(ow) for _ in range(nout)]
    out_shape = [jax.ShapeDtypeStruct((NPAD, ow), _f32) for _ in range(nout)]

    def call(args):
        return pl.pallas_call(
            kern,
            grid=(GRID,),
            in_specs=in_specs_one(),
            out_specs=out_specs,
            out_shape=out_shape,
        )(*args)

    return call


def _final_kernel(h5c0, h5c1, eac, sic, wbc, bc, wn, bnr,
                  h5t0, h5t1, eat, sit, wbt, bt, wsf, bsr, onccl, ostrat):
    c5 = (h5c0[...] + h5c1[...]
          + _bdot(eac[...][:, 0:16], wbc[...])) * sic[...] + bc[...]
    t5 = (h5t0[...] + h5t1[...]
          + _bdot(eat[...][:, 0:16], wbt[...])) * sit[...] + bt[...]
    onccl[...] = _bdot(c5, wn[...]) + bnr[...]
    ostrat[...] = _bdot(t5, wsf[...]) + bsr[...]


def _final(*args):
    return pl.pallas_call(
        _final_kernel,
        grid=(GRID,),
        in_specs=[
            _row_spec(64), _row_spec(64), _row_spec(16),
            pl.BlockSpec((RB, 1), lambda i: (i, 0)),
            _full_spec((16, 64)), _full_spec((1, 64)),
            _full_spec((64, 1)), _full_spec((1, 1)),
            _row_spec(64), _row_spec(64), _row_spec(16),
            pl.BlockSpec((RB, 1), lambda i: (i, 0)),
            _full_spec((16, 64)), _full_spec((1, 64)),
            _full_spec((64, 8)), _full_spec((1, 8)),
        ],
        out_specs=[pl.BlockSpec((RB, 1), lambda i: (i, 0)), _row_spec(8)],
        out_shape=[jax.ShapeDtypeStruct((NPAD, 1), _f32),
                   jax.ShapeDtypeStruct((NPAD, 8), _f32)],
    )(*args)


def kernel(cfeats, cedge_feats, ctypes, tfeats, tedge_feats, c_src, c_dst,
           t_src, t_dst, op_emb, c_Ws, c_bs, t_Ws, t_bs, Wn, bn, Ws_final,
           bs_final):
    layer_ep0 = _make_layer_tc(2, 128, HID, True, None)
    layer_mid = _make_layer_tc(8, HID, HID, True, None)
    layer_ep3 = _make_layer_tc(8, HID, HID, True, 64)
    seg128 = _make_segsum(BW, 2, False)
    seg512 = _make_segsum(BW, 8, False)
    seg64 = _make_segsum(BW, 1, True)

    i32 = jnp.int32
    idx = {}
    idxs = {}
    for g, (s, d) in (("c", (c_src, c_dst)), ("t", (t_src, t_dst))):
        s32, d32 = s.astype(i32), d.astype(i32)
        idx[g] = (s32.reshape(NS, ROWS_PER_TILE, C),
                  d32.reshape(NS, ROWS_PER_TILE, C))
        idxs[g] = (s32.reshape(2, NS, ROWS_PER_TILE // 2, C),
                   d32.reshape(2, NS, ROWS_PER_TILE // 2, C))

    od_c, id_c, ea_c, od_t, id_t, ea_t = _build_deg_eagg()(
        *idx["c"], cedge_feats, *idx["t"], tedge_feats)

    pad = ((0, NPAD - N), (0, 0))
    fs0ca, fs0cb, fs0ta, fs0tb, sic, soc, sit, sot = _prep(
        od_c, id_c, od_t, id_t, jnp.pad(cfeats, pad),
        jnp.pad(ctypes.astype(i32).reshape(N, 1), pad),
        op_emb, jnp.pad(tfeats, pad))

    ea = dict(c=ea_c, t=ea_t)
    si = dict(c=sic, t=sit)
    so = dict(c=soc, t=sot)
    Ws = dict(c=c_Ws, t=t_Ws)
    bs = dict(c=c_bs, t=t_bs)
    fs = dict(c=(fs0ca, fs0cb), t=(fs0ta, fs0tb))

    h1 = dict((g, seg128(*fs[g], *idx[g])) for g in ("c", "t"))
    for g in ("c", "t"):
        W = Ws[g][0]
        fs[g] = layer_ep0((*h1[g], ea[g], si[g], so[g], W[0:128], W[128:144],
                           bs[g][0].reshape(1, HID)))

    for i in (1, 2):
        h = dict((g, seg512(*fs[g], *idx[g])) for g in ("c", "t"))
        for g in ("c", "t"):
            W = Ws[g][i]
            fs[g] = layer_mid((*h[g], ea[g], si[g], so[g], W[0:HID], W[HID:],
                               bs[g][i].reshape(1, HID)))

    h = dict((g, seg512(*fs[g], *idx[g])) for g in ("c", "t"))
    z = {}
    for g in ("c", "t"):
        W = Ws[g][3]
        (z[g],) = layer_ep3((*h[g], ea[g], si[g], so[g], W[0:HID], W[HID:],
                             bs[g][3].reshape(1, HID), Ws[g][4][0:HID]))

    h5 = dict((g, seg64(z[g], *idxs[g])) for g in ("c", "t"))
    onccl, ostrat = _final(
        *h5["c"], ea_c, sic, c_Ws[4][HID:], c_bs[4].reshape(1, 64),
        Wn, (bn + jnp.zeros((1, 1), _f32)),
        *h5["t"], ea_t, sit, t_Ws[4][HID:], t_bs[4].reshape(1, 64),
        Ws_final, bs_final.reshape(1, 8))
    return (onccl.reshape(NPAD)[:N], ostrat[:N])
